# Initial kernel scaffold; baseline (speedup 1.0000x reference)
import functools
import math

import jax
import jax.numpy as jnp
from jax import lax
from jax.experimental import pallas as pl
from jax.experimental.pallas import tpu as pltpu
from jax.experimental.pallas import tpu_sc as plsc

EMB = 32
SCALE = math.sqrt(EMB)
LANES = 16

_info = plsc.get_sparse_core_info()
_NC, _NS = _info.num_cores, _info.num_subcores
NW = _NC * _NS

NBUF = 4


@functools.partial(jax.jit, static_argnames=("chunk",))
def _embed_flat(tokens_flat, table, chunk):
    B = tokens_flat.shape[0]
    bpw = B // NW
    nchunk = bpw // chunk
    ngroup = nchunk // NBUF

    mesh = plsc.VectorSubcoreMesh(core_axis_name="c", subcore_axis_name="s")

    @functools.partial(
        pl.kernel,
        mesh=mesh,
        out_type=jax.ShapeDtypeStruct((B, EMB), jnp.float32),
        scratch_types=[
            pltpu.VMEM((NBUF, chunk), jnp.int32),
            pltpu.VMEM((NBUF, chunk, EMB), jnp.float32),
        ]
        + [pltpu.SemaphoreType.DMA] * (2 * NBUF),
    )
    def body(tok_hbm, table_hbm, out_hbm, idx_v, rows_v, *sems):
        gsem = sems[:NBUF]
        osem = sems[NBUF:]
        wid = lax.axis_index("s") * _NC + lax.axis_index("c")
        base = wid * bpw

        def issue_gather(i, b):
            pltpu.sync_copy(tok_hbm.at[pl.ds(base + i * chunk, chunk)],
                            idx_v.at[b])
            pltpu.async_copy(table_hbm.at[idx_v.at[b]], rows_v.at[b], gsem[b])

        def wait_gather(b):
            pltpu.make_async_copy(table_hbm.at[idx_v.at[b]], rows_v.at[b],
                                  gsem[b]).wait()

        def scale_rows(b):
            def srow(j, _):
                for h in range(EMB // LANES):
                    sl = pl.ds(h * LANES, LANES)
                    rows_v[b, j, sl] = rows_v[b, j, sl] * SCALE
                return ()

            lax.fori_loop(0, chunk, srow, (), unroll=4)

        for b in range(NBUF):
            issue_gather(b, b)

        def group(g, _):
            for b in range(NBUF):
                i = g * NBUF + b
                dst = out_hbm.at[pl.ds(base + i * chunk, chunk)]
                wait_gather(b)
                scale_rows(b)
                pltpu.async_copy(rows_v.at[b], dst, osem[b])
                pltpu.make_async_copy(rows_v.at[b], dst, osem[b]).wait()

                @pl.when(g + 1 < ngroup)
                def _issue_next():
                    issue_gather(i + NBUF, b)

            return ()

        lax.fori_loop(0, ngroup, group, ())

    return body(tokens_flat, table)


def kernel(tokens, table):
    B = tokens.shape[0] * tokens.shape[1]
    tokens_flat = tokens.reshape(B).astype(jnp.int32)
    out = _embed_flat(tokens_flat, table, chunk=512)
    return out.reshape(tokens.shape[0], tokens.shape[1], EMB)


# speedup vs baseline: 4.0229x; 4.0229x over previous
import functools
import math

import jax
import jax.numpy as jnp
from jax import lax
from jax.experimental import pallas as pl
from jax.experimental.pallas import tpu as pltpu
from jax.experimental.pallas import tpu_sc as plsc

EMB = 32
SCALE = math.sqrt(EMB)
LANES = 16
IW = 128

_info = plsc.get_sparse_core_info()
_NC, _NS = _info.num_cores, _info.num_subcores
NW = _NC * _NS

NBUF = 4


@functools.partial(jax.jit, static_argnames=("chunk",))
def _embed_flat(tokens_2d, table, chunk):
    B = tokens_2d.shape[0] * IW
    bpw = B // NW
    K = chunk // IW
    nchunk = bpw // chunk
    ngroup = nchunk // NBUF

    mesh = plsc.VectorSubcoreMesh(core_axis_name="c", subcore_axis_name="s")

    @functools.partial(
        pl.kernel,
        mesh=mesh,
        compiler_params=pltpu.CompilerParams(use_tc_tiling_on_sc=False),
        out_type=jax.ShapeDtypeStruct((B, EMB), jnp.float32),
        scratch_types=[
            pltpu.VMEM((NBUF, K, IW), jnp.int32),
            pltpu.VMEM((NBUF, chunk, EMB), jnp.float32),
        ]
        + [pltpu.SemaphoreType.DMA] * (2 * NBUF),
    )
    def body(tok_hbm, table_hbm, out_hbm, idx_v, rows_v, *sems):
        gsem = sems[:NBUF]
        osem = sems[NBUF:]
        wid = lax.axis_index("s") * _NC + lax.axis_index("c")
        base = wid * bpw
        tbase = wid * (bpw // IW)

        def issue_gather(i, b):
            pltpu.sync_copy(tok_hbm.at[pl.ds(tbase + i * K, K)], idx_v.at[b])
            for j in range(K):
                pltpu.async_copy(table_hbm.at[idx_v.at[b, j]],
                                 rows_v.at[b, pl.ds(j * IW, IW)], gsem[b])

        def wait_gather(b):
            for j in range(K):
                pltpu.make_async_copy(table_hbm.at[idx_v.at[b, j]],
                                      rows_v.at[b, pl.ds(j * IW, IW)],
                                      gsem[b]).wait()

        def scale_rows(b):
            def srow(j, _):
                for h in range(EMB // LANES):
                    sl = pl.ds(h * LANES, LANES)
                    rows_v[b, j, sl] = rows_v[b, j, sl] * SCALE
                return ()

            lax.fori_loop(0, chunk, srow, (), unroll=4)

        for b in range(NBUF):
            issue_gather(b, b)

        def group(g, _):
            for b in range(NBUF):
                i = g * NBUF + b
                dst = out_hbm.at[pl.ds(base + i * chunk, chunk)]
                wait_gather(b)
                scale_rows(b)
                pltpu.async_copy(rows_v.at[b], dst, osem[b])
                pltpu.make_async_copy(rows_v.at[b], dst, osem[b]).wait()

                @pl.when(g + 1 < ngroup)
                def _issue_next():
                    issue_gather(i + NBUF, b)

            return ()

        lax.fori_loop(0, ngroup, group, ())

    return body(tokens_2d, table)


def kernel(tokens, table):
    B = tokens.shape[0] * tokens.shape[1]
    tokens_2d = tokens.reshape(B // IW, IW).astype(jnp.int32)
    out = _embed_flat(tokens_2d, table, chunk=512)
    return out.reshape(tokens.shape[0], tokens.shape[1], EMB)


# speedup vs baseline: 4.0672x; 1.0110x over previous
import functools
import math

import jax
import jax.numpy as jnp
from jax import lax
from jax.experimental import pallas as pl
from jax.experimental.pallas import tpu as pltpu
from jax.experimental.pallas import tpu_sc as plsc

EMB = 32
SCALE = math.sqrt(EMB)
LANES = 16
IW = 128

_info = plsc.get_sparse_core_info()
_NC, _NS = _info.num_cores, _info.num_subcores
NW = _NC * _NS

NBUF = 4
NOBUF = 2


@functools.partial(jax.jit, static_argnames=("chunk",))
def _embed_flat(tokens_2d, table, chunk):
    B = tokens_2d.shape[0] * IW
    bpw = B // NW
    K = chunk // IW
    nchunk = bpw // chunk
    ngroup = nchunk // NBUF

    mesh = plsc.VectorSubcoreMesh(core_axis_name="c", subcore_axis_name="s")

    @functools.partial(
        pl.kernel,
        mesh=mesh,
        compiler_params=pltpu.CompilerParams(use_tc_tiling_on_sc=False),
        out_type=jax.ShapeDtypeStruct((B, EMB), jnp.float32),
        scratch_types=[
            pltpu.VMEM((NBUF, K, IW), jnp.int32),
            pltpu.VMEM((NBUF, chunk, EMB), jnp.float32),
            pltpu.VMEM((NOBUF, chunk, EMB), jnp.float32),
        ]
        + [pltpu.SemaphoreType.DMA] * (2 * NBUF + NOBUF),
    )
    def body(tok_hbm, table_hbm, out_hbm, idx_v, rows_v, outb_v, *sems):
        gsem = sems[:NBUF]
        isem = sems[NBUF:2 * NBUF]
        osem = sems[2 * NBUF:]
        wid = lax.axis_index("s") * _NC + lax.axis_index("c")
        base = wid * bpw
        tbase = wid * (bpw // IW)

        def idx_copy(i, b):
            return pltpu.make_async_copy(
                tok_hbm.at[pl.ds(tbase + i * K, K)], idx_v.at[b], isem[b])

        def issue_gather(b):
            for j in range(K):
                pltpu.async_copy(table_hbm.at[idx_v.at[b, j]],
                                 rows_v.at[b, pl.ds(j * IW, IW)], gsem[b])

        def wait_gather(b):
            for j in range(K):
                pltpu.make_async_copy(table_hbm.at[idx_v.at[b, j]],
                                      rows_v.at[b, pl.ds(j * IW, IW)],
                                      gsem[b]).wait()

        def scale_rows(b, c):
            def srow(j, _):
                for h in range(EMB // LANES):
                    sl = pl.ds(h * LANES, LANES)
                    outb_v[c, j, sl] = rows_v[b, j, sl] * SCALE
                return ()

            lax.fori_loop(0, chunk, srow, (), unroll=8)

        def out_copy(i, c):
            return pltpu.make_async_copy(
                outb_v.at[c], out_hbm.at[pl.ds(base + i * chunk, chunk)],
                osem[c])

        for b in range(NBUF):
            idx_copy(b, b).start()
        for b in range(NBUF):
            idx_copy(b, b).wait()
            issue_gather(b)

        def group(g, _):
            for b in range(NBUF):
                i = g * NBUF + b
                c = b % NOBUF
                wait_gather(b)

                @pl.when(i + NBUF < nchunk)
                def _prefetch_idx():
                    idx_copy(i + NBUF, b).start()

                @pl.when(i >= NOBUF)
                def _drain_out():
                    out_copy(i - NOBUF, c).wait()

                scale_rows(b, c)
                out_copy(i, c).start()

                @pl.when(i + NBUF < nchunk)
                def _issue_next():
                    idx_copy(i + NBUF, b).wait()
                    issue_gather(b)

            return ()

        lax.fori_loop(0, ngroup, group, ())

        for c in range(NOBUF):
            out_copy(nchunk - NOBUF + c, (nchunk - NOBUF + c) % NOBUF).wait()

    return body(tokens_2d, table)


def kernel(tokens, table):
    B = tokens.shape[0] * tokens.shape[1]
    tokens_2d = tokens.reshape(B // IW, IW).astype(jnp.int32)
    out = _embed_flat(tokens_2d, table, chunk=512)
    return out.reshape(tokens.shape[0], tokens.shape[1], EMB)
